# baseline (speedup 1.0000x reference)
import math

import jax
import jax.numpy as jnp
from jax import lax
from jax.experimental import pallas as pl
from jax.experimental.pallas import tpu as pltpu
from jax.experimental.pallas import tpu_sc as plsc

B = 4
S = 4096
D = 2048
K = 512

SBLK = 512
NCHUNK = S // SBLK
SC_CORES = 2
SC_SUBCORES = 16
NW = SC_CORES * SC_SUBCORES
ROWS_PER_W = (B * K) // NW
GCHUNK = 16


def _skey(s):
    b32 = lax.bitcast_convert_type(s + 0.0, jnp.int32)
    return b32 ^ (jnp.right_shift(b32, 31) & jnp.int32(0x7FFFFFFF))


def _rank_body(row_ref, col_ref, o_ref):
    krow = _skey(row_ref[0])
    irow = lax.broadcasted_iota(jnp.int32, (1, S), 1)
    r_col = lax.broadcasted_iota(jnp.int32, (K, 1), 0)

    def count_step(ci, acc):
        kc = _skey(col_ref[0, pl.ds(ci * SBLK, SBLK), :])
        beats = (kc > krow).astype(jnp.int32)
        return acc + jnp.sum(beats, axis=0, keepdims=True)

    cnt_gt = lax.fori_loop(0, NCHUNK, count_step,
                           jnp.zeros((1, S), jnp.int32))

    def invert(rank):
        hit = rank == r_col
        return jnp.sum(jnp.where(hit, irow, 0), axis=1, keepdims=True)

    def fast(_):
        return invert(cnt_gt)

    def slow(_):
        def tie_step(ci, acc):
            kc = _skey(col_ref[0, pl.ds(ci * SBLK, SBLK), :])
            jc = lax.broadcasted_iota(jnp.int32, (SBLK, 1), 0) + ci * SBLK
            tie = ((kc == krow) & (jc < irow)).astype(jnp.int32)
            return acc + jnp.sum(tie, axis=0, keepdims=True)

        rank = lax.fori_loop(0, NCHUNK, tie_step, cnt_gt)
        return invert(rank)

    no_ties = jnp.sum(cnt_gt) == jnp.int32(S * (S - 1) // 2)
    acc = lax.cond(no_ties, fast, slow, 0)
    o_ref[0] = acc + pl.program_id(0) * S


def _gather_body(table_ref, idx_ref, out_ref, idx_v, rows_v0, rows_v1, sem0, sem1):
    wid = lax.axis_index("s") * SC_CORES + lax.axis_index("c")
    base = wid * ROWS_PER_W
    nch = ROWS_PER_W // GCHUNK
    for ch in range(nch):
        off = base + ch * GCHUNK
        pltpu.sync_copy(idx_ref.at[pl.ds(off, GCHUNK)], idx_v)
        pltpu.async_copy(table_ref.at[idx_v], rows_v0, sem0).wait()
        pltpu.sync_copy(rows_v0, out_ref.at[pl.ds(off, GCHUNK)])


_rank_call = pl.pallas_call(
    _rank_body,
    grid=(B,),
    in_specs=[
        pl.BlockSpec((1, 1, S), lambda b: (b, 0, 0)),
        pl.BlockSpec((1, S, 1), lambda b: (b, 0, 0)),
    ],
    out_specs=pl.BlockSpec((1, K, 1), lambda b: (b, 0, 0)),
    out_shape=jax.ShapeDtypeStruct((B, K, 1), jnp.int32),
)


def _gather_call():
    return pl.kernel(
        _gather_body,
        out_type=jax.ShapeDtypeStruct((B * K, D), jnp.float32),
        mesh=plsc.VectorSubcoreMesh(
            core_axis_name="c", subcore_axis_name="s",
            num_cores=SC_CORES, num_subcores=SC_SUBCORES,
        ),
        scratch_types=[
            pltpu.VMEM((GCHUNK,), jnp.int32),
            pltpu.VMEM((GCHUNK, D), jnp.float32),
            pltpu.VMEM((GCHUNK, D), jnp.float32),
            pltpu.SemaphoreType.DMA,
            pltpu.SemaphoreType.DMA,
        ],
    )


def kernel(x, gamma, beta, W, b):
    mean = jnp.mean(x, axis=-1, keepdims=True)
    var = jnp.var(x, axis=-1, keepdims=True)
    xn = (x - mean) / jnp.sqrt(var + 1e-5) * gamma + beta
    scores = ((jnp.einsum('bsd,do->bso', xn, W) + b) / math.sqrt(2))[..., 0]
    idx = _rank_call(scores.reshape(B, 1, S), scores.reshape(B, S, 1))
    rows = _gather_call()(x.reshape(B * S, D), idx.reshape(B * K))
    return rows.reshape(B, K, D)


# speedup vs baseline: 1.0135x; 1.0135x over previous
import math

import jax
import jax.numpy as jnp
from jax import lax
from jax.experimental import pallas as pl
from jax.experimental.pallas import tpu as pltpu
from jax.experimental.pallas import tpu_sc as plsc

B = 4
S = 4096
D = 2048
K = 512

SBLK = 512
NCHUNK = S // SBLK
SC_CORES = 2
SC_SUBCORES = 16
NW = SC_CORES * SC_SUBCORES
ROWS_PER_W = (B * K) // NW
GCHUNK = 16


def _skey(s):
    b32 = lax.bitcast_convert_type(s + 0.0, jnp.int32)
    return b32 ^ (jnp.right_shift(b32, 31) & jnp.int32(0x7FFFFFFF))


def _rank_body(row_ref, col_ref, o_ref):
    krow = _skey(row_ref[0])
    irow = lax.broadcasted_iota(jnp.int32, (1, S), 1)
    r_col = lax.broadcasted_iota(jnp.int32, (K, 1), 0)

    def count_step(ci, acc):
        kc = _skey(col_ref[0, pl.ds(ci * SBLK, SBLK), :])
        beats = (kc > krow).astype(jnp.int32)
        return acc + jnp.sum(beats, axis=0, keepdims=True)

    cnt_gt = lax.fori_loop(0, NCHUNK, count_step,
                           jnp.zeros((1, S), jnp.int32))

    def invert(rank):
        hit = rank == r_col
        return jnp.sum(jnp.where(hit, irow, 0), axis=1, keepdims=True)

    def fast(_):
        return invert(cnt_gt)

    def slow(_):
        def tie_step(ci, acc):
            kc = _skey(col_ref[0, pl.ds(ci * SBLK, SBLK), :])
            jc = lax.broadcasted_iota(jnp.int32, (SBLK, 1), 0) + ci * SBLK
            tie = ((kc == krow) & (jc < irow)).astype(jnp.int32)
            return acc + jnp.sum(tie, axis=0, keepdims=True)

        rank = lax.fori_loop(0, NCHUNK, tie_step, cnt_gt)
        return invert(rank)

    no_ties = jnp.sum(cnt_gt) == jnp.int32(S * (S - 1) // 2)
    acc = lax.cond(no_ties, fast, slow, 0)
    o_ref[0] = acc + pl.program_id(0) * S


def _gather_body(table_ref, idx_ref, out_ref, idx_v, rows_v0, rows_v1, sem0, sem1):
    wid = lax.axis_index("s") * SC_CORES + lax.axis_index("c")
    base = wid * ROWS_PER_W
    pltpu.sync_copy(idx_ref.at[pl.ds(base, ROWS_PER_W)], idx_v)
    bufs = (rows_v0, rows_v1)
    sems = (sem0, sem1)
    nch = ROWS_PER_W // GCHUNK
    cps = [None] * nch

    def fire(ch):
        idx_c = idx_v.at[pl.ds(ch * GCHUNK, GCHUNK)]
        cps[ch] = pltpu.async_copy(table_ref.at[idx_c], bufs[ch % 2], sems[ch % 2])

    fire(0)
    for ch in range(nch):
        if ch + 1 < nch:
            fire(ch + 1)
        cps[ch].wait()
        pltpu.sync_copy(bufs[ch % 2], out_ref.at[pl.ds(base + ch * GCHUNK, GCHUNK)])


_rank_call = pl.pallas_call(
    _rank_body,
    grid=(B,),
    in_specs=[
        pl.BlockSpec((1, 1, S), lambda b: (b, 0, 0)),
        pl.BlockSpec((1, S, 1), lambda b: (b, 0, 0)),
    ],
    out_specs=pl.BlockSpec((1, K, 1), lambda b: (b, 0, 0)),
    out_shape=jax.ShapeDtypeStruct((B, K, 1), jnp.int32),
)


def _gather_call():
    return pl.kernel(
        _gather_body,
        out_type=jax.ShapeDtypeStruct((B * K, D), jnp.float32),
        mesh=plsc.VectorSubcoreMesh(
            core_axis_name="c", subcore_axis_name="s",
            num_cores=SC_CORES, num_subcores=SC_SUBCORES,
        ),
        scratch_types=[
            pltpu.VMEM((ROWS_PER_W,), jnp.int32),
            pltpu.VMEM((GCHUNK, D), jnp.float32),
            pltpu.VMEM((GCHUNK, D), jnp.float32),
            pltpu.SemaphoreType.DMA,
            pltpu.SemaphoreType.DMA,
        ],
    )


def kernel(x, gamma, beta, W, b):
    mean = jnp.mean(x, axis=-1, keepdims=True)
    var = jnp.var(x, axis=-1, keepdims=True)
    xn = (x - mean) / jnp.sqrt(var + 1e-5) * gamma + beta
    scores = ((jnp.einsum('bsd,do->bso', xn, W) + b) / math.sqrt(2))[..., 0]
    idx = _rank_call(scores.reshape(B, 1, S), scores.reshape(B, S, 1))
    rows = _gather_call()(x.reshape(B * S, D), idx.reshape(B * K))
    return rows.reshape(B, K, D)
